# baseline (speedup 1.0000x reference)
import functools

import jax
import jax.numpy as jnp
import numpy as np
from jax import lax
from jax.experimental import pallas as pl
from jax.experimental.pallas import tpu as pltpu
from jax.experimental.pallas import tpu_sc as plsc

H = 4
C = 32
D = H * C
G = 128
NC = 2
NS = 16
L = 16
NW = NC * NS

INV_SQRT_C = float(1.0 / np.sqrt(np.float32(C)))


def _proj_body(h_ref, w_ref, b_ref, o_ref):
    o_ref[...] = jnp.dot(h_ref[...], w_ref[...],
                         preferred_element_type=jnp.float32) + b_ref[...]


def _proj(h, wcat, bcat):
    n = h.shape[0]
    bn = 1000
    nsteps = n // bn
    return pl.pallas_call(
        _proj_body,
        grid=(4, nsteps),
        in_specs=[
            pl.BlockSpec((bn, D), lambda c, i: (i, 0)),
            pl.BlockSpec((D, D), lambda c, i: (0, c)),
            pl.BlockSpec((1, D), lambda c, i: (0, c)),
        ],
        out_specs=pl.BlockSpec((bn, D), lambda c, i: (c * nsteps + i, 0)),
        out_shape=jax.ShapeDtypeStruct((4 * n, D), jnp.float32),
    )(h, wcat, bcat)


def _e_body(a_ref, w_ref, o_ref):
    o_ref[...] = jnp.dot(a_ref[...], w_ref[...],
                         preferred_element_type=jnp.float32)


def _e_matmul(edge_attr, we):
    e_num, ed = edge_attr.shape
    rb = 4000
    grid = e_num // rb
    return pl.pallas_call(
        _e_body,
        grid=(grid,),
        in_specs=[
            pl.BlockSpec((rb, ed), lambda i: (i, 0)),
            pl.BlockSpec((ed, D), lambda i: (0, 0)),
        ],
        out_specs=pl.BlockSpec((rb, D), lambda i: (i, 0)),
        out_shape=jax.ShapeDtypeStruct((e_num, D), jnp.float32),
    )(edge_attr, we)


B = 32
GE = 16
NPAD = 10240
DR = NPAD * H // 128
NACC = 10752


def _sc_edge(qkvx, e, comb, n):
    e_num = e.shape[0]
    nb = e_num // B
    npw = nb // NW
    rows_per_tile = NACC // NS
    mesh = plsc.VectorSubcoreMesh(core_axis_name="c", subcore_axis_name="s",
                                  num_cores=NC, num_subcores=NS)

    @functools.partial(
        pl.kernel,
        out_type=(jax.ShapeDtypeStruct((NC, NACC, D), jnp.float32),
                  jax.ShapeDtypeStruct((NC, NPAD, L), jnp.float32)),
        mesh=mesh,
        compiler_params=pltpu.CompilerParams(use_tc_tiling_on_sc=False),
        scratch_types=[
            pltpu.VMEM_SHARED((NACC, D), jnp.float32),
            pltpu.VMEM_SHARED((NPAD, L), jnp.float32),
            pltpu.VMEM((1, 2 * B), jnp.int32),
            pltpu.VMEM((1, 2 * B), jnp.int32),
            pltpu.VMEM((1, 3 * B), jnp.int32),
            pltpu.VMEM((1, 3 * B), jnp.int32),
            pltpu.VMEM((1, 2 * B), jnp.int32),
            pltpu.VMEM((1, 2 * B), jnp.int32),
            pltpu.VMEM((3 * B, D), jnp.float32),
            pltpu.VMEM((3 * B, D), jnp.float32),
            pltpu.VMEM((B, D), jnp.float32),
            pltpu.VMEM((B, D), jnp.float32),
            pltpu.VMEM((B, L), jnp.float32),
            pltpu.VMEM((B, L), jnp.float32),
            pltpu.SemaphoreType.DMA,
            pltpu.SemaphoreType.DMA,
            pltpu.SemaphoreType.DMA,
            pltpu.SemaphoreType.DMA,
            pltpu.SemaphoreType.DMA,
            pltpu.SemaphoreType.DMA,
        ],
    )
    def body(qkvx_hbm, e_hbm, comb_hbm, out_hbm, den_hbm,
             acc, dacc, li0, li1, gx0, gx1, sx0, sx1, qv0, qv1, ev0, ev1,
             dn0, dn1, si0, si1, sg0, sg1, ss0, ss1):
        cid = lax.axis_index("c")
        sid = lax.axis_index("s")
        w = sid * NC + cid
        w0 = w * npw

        LI = (li0, li1)
        GX = (gx0, gx1)
        SX = (sx0, sx1)
        QV = (qv0, qv1)
        EV = (ev0, ev1)
        DN = (dn0, dn1)
        SI = (si0, si1)
        SG = (sg0, sg1)
        SS = (ss0, ss1)

        lane = lax.iota(jnp.int32, L)
        zero16 = jnp.zeros((L,), jnp.float32)
        lsplat = jnp.full((L,), L, jnp.int32)
        rot_idx = [lax.rem(lane + sh, lsplat) for sh in (8, 4, 2, 1)]
        gdn = lax.GatherDimensionNumbers(offset_dims=(),
                                         collapsed_slice_dims=(0,),
                                         start_index_map=(0,))

        def _allsum(x):
            for idx in rot_idx:
                x = x + lax.gather(x, idx[:, None], gdn, (1,),
                                   mode=lax.GatherScatterMode.PROMISE_IN_BOUNDS)
            return x

        m_eq0 = lane == 0
        m_eq2 = lane == 2
        m_lt2 = lane < 2
        m_lt4 = lane < H

        def issue_idx(p, kblk):
            pltpu.async_copy(comb_hbm.at[pl.ds((w0 + kblk) * 2 * B, 2 * B)],
                             LI[p].at[0], SI[p])

        def wait_idx(p):
            pltpu.make_async_copy(comb_hbm.at[pl.ds(0, 2 * B)],
                                  LI[p].at[0], SI[p]).wait()

        def compute_idx(p):
            for ccol in range(B // L):
                s16 = LI[p][0, pl.ds(ccol * L, L)]
                d16 = LI[p][0, pl.ds(B + ccol * L, L)]
                GX[p][0, pl.ds(ccol * L, L)] = d16
                GX[p][0, pl.ds(B + ccol * L, L)] = s16 + n
                GX[p][0, pl.ds(2 * B + ccol * L, L)] = s16 + 2 * n
                SX[p][0, pl.ds(ccol * L, L)] = d16

        def issue_gathers(p, kblk):
            base = (w0 + kblk) * B
            pltpu.async_copy(qkvx_hbm.at[GX[p].at[0]], QV[p], SG[p])
            pltpu.async_copy(e_hbm.at[pl.ds(base, B)], EV[p], SG[p])

        def wait_gathers(p):
            pltpu.make_async_copy(e_hbm.at[pl.ds(0, 3 * B)], QV[p],
                                  SG[p]).wait()
            pltpu.make_async_copy(e_hbm.at[pl.ds(0, B)], EV[p],
                                  SG[p]).wait()

        def issue_scatters(p):
            dstrow = SX[p].at[0, pl.ds(0, B)]
            pltpu.async_copy(QV[p].at[pl.ds(0, B)], acc.at[dstrow],
                             SS[p], add=True)
            pltpu.async_copy(DN[p], dacc.at[dstrow], SS[p], add=True)

        def wait_scatters(p):
            pltpu.make_async_copy(e_hbm.at[pl.ds(0, B)],
                                  QV[p].at[pl.ds(0, B)], SS[p]).wait()
            pltpu.make_async_copy(den_hbm.at[0, pl.ds(0, B)],
                                  DN[p], SS[p]).wait()

        def compute(p):
            qv, ev, liref, dnv = QV[p], EV[p], LI[p], DN[p]

            @pl.loop(0, B // GE)
            def _(g):
                for j in range(GE):
                    i = g * GE + j
                    prods = []
                    ees = []
                    for cc in range(D // L):
                        qq = qv[i, pl.ds(cc * L, L)]
                        kk = qv[B + i, pl.ds(cc * L, L)]
                        ee = ev[i, pl.ds(cc * L, L)]
                        ees.append(ee)
                        prods.append(qq * (kk + ee))
                    exs = []
                    for h in range(H):
                        sv = _allsum(prods[2 * h] + prods[2 * h + 1])
                        exs.append(jnp.exp(sv * INV_SQRT_C))
                    for h in range(H):
                        for cc in (2 * h, 2 * h + 1):
                            vvv = qv[2 * B + i, pl.ds(cc * L, L)]
                            qv[i, pl.ds(cc * L, L)] = (vvv + ees[cc]) * exs[h]
                    dv = jnp.where(m_lt2,
                                   jnp.where(m_eq0, exs[0], exs[1]),
                                   jnp.where(m_eq2, exs[2], exs[3]))
                    dnv[i, pl.ds(0, L)] = jnp.where(m_lt4, dv, 0.0)

        @pl.loop(0, B)
        def _(r):
            for ccol in range(D // L):
                qv0[r, pl.ds(ccol * L, L)] = zero16

        base_row = sid * rows_per_tile
        for t in range(rows_per_tile // B):
            pltpu.sync_copy(qv0.at[pl.ds(0, B)],
                            acc.at[pl.ds(base_row + t * B, B)])
        @pl.loop(0, B)
        def _(r):
            dn0[r, pl.ds(0, L)] = zero16
        dbase = sid * (NPAD // NS)
        for t in range(NPAD // NS // B):
            pltpu.sync_copy(dn0, dacc.at[pl.ds(dbase + t * B, B)])
        plsc.subcore_barrier()

        pltpu.sync_copy(comb_hbm.at[pl.ds(w0 * 2 * B, 2 * B)], li0.at[0])
        compute_idx(0)
        issue_gathers(0, 0)
        issue_idx(1, 1)

        @pl.loop(0, (npw - 1) // 2)
        def _(tp):
            k0 = tp * 2

            @pl.when(tp > 0)
            def _():
                wait_scatters(1)
            wait_idx(1)
            compute_idx(1)
            issue_gathers(1, k0 + 1)
            wait_gathers(0)
            compute(0)
            issue_idx(0, k0 + 2)
            issue_scatters(0)

            wait_scatters(0)
            wait_idx(0)
            compute_idx(0)
            issue_gathers(0, k0 + 2)
            wait_gathers(1)
            compute(1)

            @pl.when(k0 + 3 <= npw - 1)
            def _():
                issue_idx(1, k0 + 3)
            issue_scatters(1)

        wait_scatters(1)
        wait_gathers(0)
        compute(0)
        issue_scatters(0)
        wait_scatters(0)
        plsc.subcore_barrier()

        pltpu.sync_copy(acc.at[pl.ds(base_row, rows_per_tile)],
                        out_hbm.at[cid, pl.ds(base_row, rows_per_tile)])
        pltpu.sync_copy(dacc.at[pl.ds(dbase, NPAD // NS)],
                        den_hbm.at[cid, pl.ds(dbase, NPAD // NS)])

    return body(qkvx, e, comb)


def _gate_body(p_ref, d_ref, xr_ref, wba_ref, wbb_ref, o_ref):
    outp = p_ref[0] + p_ref[1]
    bn = outp.shape[0]
    den = (d_ref[0] + d_ref[1])[:, 0:H]
    div = jnp.concatenate(
        [jnp.broadcast_to(den[:, h:h + 1], (bn, C)) for h in range(H)], axis=1)
    out = outp / (div + 1e-16)
    xr = xr_ref[...]
    lin = (jnp.sum(xr * wba_ref[...], axis=1, keepdims=True)
           + jnp.sum(out * wbb_ref[...], axis=1, keepdims=True))
    beta = jax.nn.sigmoid(lin)
    o_ref[...] = beta * xr + (1.0 - beta) * out


def _gate(parts, den3, qkvx, wba, wbb, n):
    bn = 1000
    grid = n // bn
    return pl.pallas_call(
        _gate_body,
        grid=(grid,),
        in_specs=[
            pl.BlockSpec((NC, bn, D), lambda i: (0, i, 0)),
            pl.BlockSpec((NC, bn, L), lambda i: (0, i, 0)),
            pl.BlockSpec((bn, D), lambda i: (3 * grid + i, 0)),
            pl.BlockSpec((1, D), lambda i: (0, 0)),
            pl.BlockSpec((1, D), lambda i: (0, 0)),
        ],
        out_specs=pl.BlockSpec((bn, D), lambda i: (i, 0)),
        out_shape=jax.ShapeDtypeStruct((n, D), jnp.float32),
    )(parts, den3, qkvx, wba, wbb)


def _pool_body(h_ref, b_ref, w1_ref, b1_ref, w2_ref, b2_ref, o_ref,
               acc_s, cnt_s):
    i = pl.program_id(0)

    @pl.when(i == 0)
    def _():
        acc_s[...] = jnp.zeros_like(acc_s)
        cnt_s[...] = jnp.zeros_like(cnt_s)

    gi = lax.broadcasted_iota(jnp.int32, (G, 1), 0).astype(jnp.float32)
    bvals = b_ref[0]
    mask = (gi == bvals).astype(jnp.float32)
    acc_s[...] += jnp.dot(mask, h_ref[...], preferred_element_type=jnp.float32)
    cnt_s[...] += jnp.sum(mask, axis=1, keepdims=True)

    @pl.when(i == pl.num_programs(0) - 1)
    def _():
        pooled = acc_s[...] / jnp.maximum(cnt_s[...], 1.0)
        h1 = jnp.maximum(
            jnp.dot(pooled, w1_ref[...], preferred_element_type=jnp.float32)
            + b1_ref[...], 0.0)
        o_ref[...] = (jnp.dot(h1, w2_ref[...],
                              preferred_element_type=jnp.float32)
                      + b2_ref[...])


def _pool(h, bidx3, w1, b1, w2, b2):
    n = h.shape[0]
    bn = 1000
    grid = n // bn
    return pl.pallas_call(
        _pool_body,
        grid=(grid,),
        in_specs=[
            pl.BlockSpec((bn, D), lambda i: (i, 0)),
            pl.BlockSpec((1, 1, bn), lambda i: (i, 0, 0)),
            pl.BlockSpec((D, 32), lambda i: (0, 0)),
            pl.BlockSpec((1, 32), lambda i: (0, 0)),
            pl.BlockSpec((32, 10), lambda i: (0, 0)),
            pl.BlockSpec((1, 10), lambda i: (0, 0)),
        ],
        out_specs=pl.BlockSpec((G, 10), lambda i: (0, 0)),
        out_shape=jax.ShapeDtypeStruct((G, 10), jnp.float32),
        scratch_shapes=[
            pltpu.VMEM((G, D), jnp.float32),
            pltpu.VMEM((G, 1), jnp.float32),
        ],
    )(h, bidx3, w1, b1, w2, b2)


def kernel(x, edge_attr, params, edge_index, batch_index):
    src = edge_index[0]
    dst = edge_index[1]
    n = x.shape[0]
    e_num = src.shape[0]
    nb = e_num // B
    comb = jnp.stack([src.reshape(nb, B), dst.reshape(nb, B)],
                     axis=1).reshape(-1)

    h = x
    for lp in params["layers"]:
        wcat = jnp.concatenate(
            [lp["Wq"], lp["Wk"], lp["Wv"], lp["Ws"]], axis=1)
        bcat = jnp.concatenate(
            [lp["bq"], lp["bk"], lp["bv"], lp["bs"]]).reshape(1, 4 * D)
        wb = lp["Wb"][:, 0]
        wba = (wb[0:D] + wb[2 * D:3 * D]).reshape(1, D)
        wbb = (wb[D:2 * D] - wb[2 * D:3 * D]).reshape(1, D)

        qkvx = _proj(h, wcat, bcat)
        e = _e_matmul(edge_attr, lp["We"])
        parts, dens = _sc_edge(qkvx, e, comb, n)
        h = _gate(parts, dens, qkvx, wba, wbb, n)

    bn = 1000
    bidx3 = batch_index.astype(jnp.float32).reshape(n // bn, 1, bn)
    return _pool(h, bidx3, params["ro_W1"], params["ro_b1"].reshape(1, 32),
                 params["ro_W2"], params["ro_b2"].reshape(1, 10))


# speedup vs baseline: 1.7825x; 1.7825x over previous
import functools

import jax
import jax.numpy as jnp
import numpy as np
from jax import lax
from jax.experimental import pallas as pl
from jax.experimental.pallas import tpu as pltpu
from jax.experimental.pallas import tpu_sc as plsc

H = 4
C = 32
D = H * C
G = 128
NC = 2
NS = 16
L = 16
NW = NC * NS

INV_SQRT_C = float(1.0 / np.sqrt(np.float32(C)))


def _proj_body(h_ref, w_ref, b_ref, o_ref):
    o_ref[...] = jnp.dot(h_ref[...], w_ref[...],
                         preferred_element_type=jnp.float32) + b_ref[...]


def _proj(h, wcat, bcat):
    n = h.shape[0]
    bn = 1000
    nsteps = n // bn
    return pl.pallas_call(
        _proj_body,
        grid=(4, nsteps),
        in_specs=[
            pl.BlockSpec((bn, D), lambda c, i: (i, 0)),
            pl.BlockSpec((D, D), lambda c, i: (0, c)),
            pl.BlockSpec((1, D), lambda c, i: (0, c)),
        ],
        out_specs=pl.BlockSpec((bn, D), lambda c, i: (c * nsteps + i, 0)),
        out_shape=jax.ShapeDtypeStruct((4 * n, D), jnp.float32),
    )(h, wcat, bcat)


def _e_body(a_ref, w_ref, o_ref):
    o_ref[...] = jnp.dot(a_ref[...], w_ref[...],
                         preferred_element_type=jnp.float32)


def _e_matmul(edge_attr, we):
    e_num, ed = edge_attr.shape
    rb = 4000
    grid = e_num // rb
    return pl.pallas_call(
        _e_body,
        grid=(grid,),
        in_specs=[
            pl.BlockSpec((rb, ed), lambda i: (i, 0)),
            pl.BlockSpec((ed, D), lambda i: (0, 0)),
        ],
        out_specs=pl.BlockSpec((rb, D), lambda i: (i, 0)),
        out_shape=jax.ShapeDtypeStruct((e_num, D), jnp.float32),
    )(edge_attr, we)


B = 32
GE = 16
NPAD = 10240
DR = NPAD * H // 128
NACC = 10752


def _sc_edge(qkvx, e, comb, n):
    e_num = e.shape[0]
    nb = e_num // B
    npw = nb // NW
    rows_per_tile = NACC // NS
    mesh = plsc.VectorSubcoreMesh(core_axis_name="c", subcore_axis_name="s",
                                  num_cores=NC, num_subcores=NS)

    @functools.partial(
        pl.kernel,
        out_type=jax.ShapeDtypeStruct((NC, NACC, D), jnp.float32),
        mesh=mesh,
        scratch_types=[
            pltpu.VMEM_SHARED((NACC, D), jnp.float32),
            pltpu.VMEM((1, 2 * B), jnp.int32),
            pltpu.VMEM((1, 2 * B), jnp.int32),
            pltpu.VMEM((1, 3 * B), jnp.int32),
            pltpu.VMEM((1, 3 * B), jnp.int32),
            pltpu.VMEM((1, 2 * B), jnp.int32),
            pltpu.VMEM((1, 2 * B), jnp.int32),
            pltpu.VMEM((3 * B, D), jnp.float32),
            pltpu.VMEM((3 * B, D), jnp.float32),
            pltpu.VMEM((B, D), jnp.float32),
            pltpu.VMEM((B, D), jnp.float32),
            pltpu.VMEM((1, B), jnp.int32),
            pltpu.VMEM((1, B), jnp.int32),
            pltpu.SemaphoreType.DMA,
            pltpu.SemaphoreType.DMA,
            pltpu.SemaphoreType.DMA,
            pltpu.SemaphoreType.DMA,
            pltpu.SemaphoreType.DMA,
            pltpu.SemaphoreType.DMA,
        ],
    )
    def body(qkvx_hbm, e_hbm, comb_hbm, out_hbm,
             acc, li0, li1, gx0, gx1, sx0, sx1, qv0, qv1, ev0, ev1,
             po0, po1, si0, si1, sg0, sg1, ss0, ss1):
        cid = lax.axis_index("c")
        sid = lax.axis_index("s")
        w = sid * NC + cid
        w0 = w * npw

        LI = (li0, li1)
        GX = (gx0, gx1)
        SX = (sx0, sx1)
        QV = (qv0, qv1)
        EV = (ev0, ev1)
        PO = (po0, po1)
        SI = (si0, si1)
        SG = (sg0, sg1)
        SS = (ss0, ss1)

        lane = lax.iota(jnp.int32, L)
        zero16 = jnp.zeros((L,), jnp.float32)
        lsplat = jnp.full((L,), L, jnp.int32)
        rot_idx = [lax.rem(lane + sh, lsplat) for sh in (8, 4, 2, 1)]
        gdn = lax.GatherDimensionNumbers(offset_dims=(),
                                         collapsed_slice_dims=(0,),
                                         start_index_map=(0,))

        def _allsum(x):
            for idx in rot_idx:
                x = x + lax.gather(x, idx[:, None], gdn, (1,),
                                   mode=lax.GatherScatterMode.PROMISE_IN_BOUNDS)
            return x

        m_eq0 = lane == 0
        m_eq2 = lane == 2
        m_lt2 = lane < 2
        m_lt4 = lane < H

        def issue_idx(p, kblk):
            pltpu.async_copy(comb_hbm.at[pl.ds((w0 + kblk) * 2 * B, 2 * B)],
                             LI[p].at[0], SI[p])

        def wait_idx(p):
            pltpu.make_async_copy(comb_hbm.at[pl.ds(0, 2 * B)],
                                  LI[p].at[0], SI[p]).wait()

        def compute_idx(p):
            for ccol in range(B // L):
                s16 = LI[p][0, pl.ds(ccol * L, L)]
                d16 = LI[p][0, pl.ds(B + ccol * L, L)]
                GX[p][0, pl.ds(ccol * L, L)] = d16
                GX[p][0, pl.ds(B + ccol * L, L)] = s16 + n
                GX[p][0, pl.ds(2 * B + ccol * L, L)] = s16 + 2 * n
                SX[p][0, pl.ds(ccol * L, L)] = d16
                SX[p][0, pl.ds(B + ccol * L, L)] = (
                    NPAD + lax.shift_right_logical(d16, 5))

        def issue_gathers(p, kblk):
            base = (w0 + kblk) * B
            pltpu.async_copy(qkvx_hbm.at[GX[p].at[0]], QV[p], SG[p])
            pltpu.async_copy(e_hbm.at[pl.ds(base, B)], EV[p], SG[p])

        def wait_gathers(p):
            pltpu.make_async_copy(e_hbm.at[pl.ds(0, 3 * B)], QV[p],
                                  SG[p]).wait()
            pltpu.make_async_copy(e_hbm.at[pl.ds(0, B)], EV[p],
                                  SG[p]).wait()

        def issue_scatters(p):
            pltpu.async_copy(QV[p].at[pl.ds(0, 2 * B)], acc.at[SX[p].at[0]],
                             SS[p], add=True)

        def wait_scatters(p):
            pltpu.make_async_copy(e_hbm.at[pl.ds(0, 2 * B)],
                                  QV[p].at[pl.ds(0, 2 * B)], SS[p]).wait()

        def compute(p):
            qv, ev, liref, pov = QV[p], EV[p], LI[p], PO[p]

            @pl.loop(0, B // GE)
            def _(g):
                dsts16 = liref[0, pl.ds(B + g * GE, GE)]
                po16 = pov[0, pl.ds(g * GE, GE)]
                colo16 = lax.shift_left(lax.bitwise_and(dsts16, 31), 2)
                b16v = colo16 - lax.bitwise_and(colo16, 12)
                pov[0, pl.ds(g * GE, GE)] = b16v
                for j in range(GE):
                    i = g * GE + j
                    prods = []
                    ees = []
                    for cc in range(D // L):
                        qq = qv[i, pl.ds(cc * L, L)]
                        kk = qv[B + i, pl.ds(cc * L, L)]
                        ee = ev[i, pl.ds(cc * L, L)]
                        ees.append(ee)
                        prods.append(qq * (kk + ee))
                    exs = []
                    for h in range(H):
                        sv = _allsum(prods[2 * h] + prods[2 * h + 1])
                        exs.append(jnp.exp(sv * INV_SQRT_C))
                    for h in range(H):
                        for cc in (2 * h, 2 * h + 1):
                            vvv = qv[2 * B + i, pl.ds(cc * L, L)]
                            qv[i, pl.ds(cc * L, L)] = (vvv + ees[cc]) * exs[h]
                    d_i = dsts16[j]
                    qv[B + i, pl.ds(po16[j], L)] = zero16
                    dv = jnp.where(m_lt2,
                                   jnp.where(m_eq0, exs[0], exs[1]),
                                   jnp.where(m_eq2, exs[2], exs[3]))
                    dv = jnp.where(m_lt4, dv, 0.0)
                    colo = lax.shift_left(lax.bitwise_and(d_i, 31), 2)
                    pp = lax.bitwise_and(colo, 12)
                    base16 = colo - pp
                    ridx = lax.bitwise_and(lane + (16 - pp), 15)
                    dvs = lax.gather(dv, ridx[:, None], gdn, (1,),
                                     mode=lax.GatherScatterMode.PROMISE_IN_BOUNDS)
                    qv[B + i, pl.ds(base16, L)] = dvs

        @pl.loop(0, B)
        def _(r):
            for ccol in range(D // L):
                qv0[r, pl.ds(ccol * L, L)] = zero16
                qv0[B + r, pl.ds(ccol * L, L)] = zero16
                qv1[B + r, pl.ds(ccol * L, L)] = zero16
        for ccol in range(B // L):
            po0[0, pl.ds(ccol * L, L)] = jnp.zeros((L,), jnp.int32)
            po1[0, pl.ds(ccol * L, L)] = jnp.zeros((L,), jnp.int32)

        base_row = sid * rows_per_tile
        for t in range(rows_per_tile // B):
            pltpu.sync_copy(qv0.at[pl.ds(0, B)],
                            acc.at[pl.ds(base_row + t * B, B)])
        plsc.subcore_barrier()

        pltpu.sync_copy(comb_hbm.at[pl.ds(w0 * 2 * B, 2 * B)], li0.at[0])
        compute_idx(0)
        issue_gathers(0, 0)
        issue_idx(1, 1)

        @pl.loop(0, (npw - 1) // 2)
        def _(tp):
            k0 = tp * 2

            @pl.when(tp > 0)
            def _():
                wait_scatters(1)
            wait_idx(1)
            compute_idx(1)
            issue_gathers(1, k0 + 1)
            wait_gathers(0)
            compute(0)
            issue_idx(0, k0 + 2)
            issue_scatters(0)

            wait_scatters(0)
            wait_idx(0)
            compute_idx(0)
            issue_gathers(0, k0 + 2)
            wait_gathers(1)
            compute(1)

            @pl.when(k0 + 3 <= npw - 1)
            def _():
                issue_idx(1, k0 + 3)
            issue_scatters(1)

        wait_scatters(1)
        wait_gathers(0)
        compute(0)
        issue_scatters(0)
        wait_scatters(0)
        plsc.subcore_barrier()

        pltpu.sync_copy(acc.at[pl.ds(base_row, rows_per_tile)],
                        out_hbm.at[cid, pl.ds(base_row, rows_per_tile)])

    return body(qkvx, e, comb)


def _gate_body(p_ref, d_ref, xr_ref, wba_ref, wbb_ref, o_ref):
    outp = p_ref[0] + p_ref[1]
    bn = outp.shape[0]
    den = d_ref[0] + d_ref[1]
    div = jnp.concatenate(
        [jnp.broadcast_to(den[:, h:h + 1], (bn, C)) for h in range(H)], axis=1)
    out = outp / (div + 1e-16)
    xr = xr_ref[...]
    lin = (jnp.sum(xr * wba_ref[...], axis=1, keepdims=True)
           + jnp.sum(out * wbb_ref[...], axis=1, keepdims=True))
    beta = jax.nn.sigmoid(lin)
    o_ref[...] = beta * xr + (1.0 - beta) * out


def _gate(parts, den3, qkvx, wba, wbb, n):
    bn = 1000
    grid = n // bn
    return pl.pallas_call(
        _gate_body,
        grid=(grid,),
        in_specs=[
            pl.BlockSpec((NC, bn, D), lambda i: (0, i, 0)),
            pl.BlockSpec((NC, bn, H), lambda i: (0, i, 0)),
            pl.BlockSpec((bn, D), lambda i: (3 * grid + i, 0)),
            pl.BlockSpec((1, D), lambda i: (0, 0)),
            pl.BlockSpec((1, D), lambda i: (0, 0)),
        ],
        out_specs=pl.BlockSpec((bn, D), lambda i: (i, 0)),
        out_shape=jax.ShapeDtypeStruct((n, D), jnp.float32),
    )(parts, den3, qkvx, wba, wbb)


def _pool_body(h_ref, b_ref, w1_ref, b1_ref, w2_ref, b2_ref, o_ref,
               acc_s, cnt_s):
    i = pl.program_id(0)

    @pl.when(i == 0)
    def _():
        acc_s[...] = jnp.zeros_like(acc_s)
        cnt_s[...] = jnp.zeros_like(cnt_s)

    gi = lax.broadcasted_iota(jnp.int32, (G, 1), 0).astype(jnp.float32)
    bvals = b_ref[0]
    mask = (gi == bvals).astype(jnp.float32)
    acc_s[...] += jnp.dot(mask, h_ref[...], preferred_element_type=jnp.float32)
    cnt_s[...] += jnp.sum(mask, axis=1, keepdims=True)

    @pl.when(i == pl.num_programs(0) - 1)
    def _():
        pooled = acc_s[...] / jnp.maximum(cnt_s[...], 1.0)
        h1 = jnp.maximum(
            jnp.dot(pooled, w1_ref[...], preferred_element_type=jnp.float32)
            + b1_ref[...], 0.0)
        o_ref[...] = (jnp.dot(h1, w2_ref[...],
                              preferred_element_type=jnp.float32)
                      + b2_ref[...])


def _pool(h, bidx3, w1, b1, w2, b2):
    n = h.shape[0]
    bn = 1000
    grid = n // bn
    return pl.pallas_call(
        _pool_body,
        grid=(grid,),
        in_specs=[
            pl.BlockSpec((bn, D), lambda i: (i, 0)),
            pl.BlockSpec((1, 1, bn), lambda i: (i, 0, 0)),
            pl.BlockSpec((D, 32), lambda i: (0, 0)),
            pl.BlockSpec((1, 32), lambda i: (0, 0)),
            pl.BlockSpec((32, 10), lambda i: (0, 0)),
            pl.BlockSpec((1, 10), lambda i: (0, 0)),
        ],
        out_specs=pl.BlockSpec((G, 10), lambda i: (0, 0)),
        out_shape=jax.ShapeDtypeStruct((G, 10), jnp.float32),
        scratch_shapes=[
            pltpu.VMEM((G, D), jnp.float32),
            pltpu.VMEM((G, 1), jnp.float32),
        ],
    )(h, bidx3, w1, b1, w2, b2)


def kernel(x, edge_attr, params, edge_index, batch_index):
    src = edge_index[0]
    dst = edge_index[1]
    n = x.shape[0]
    e_num = src.shape[0]
    nb = e_num // B
    comb = jnp.stack([src.reshape(nb, B), dst.reshape(nb, B)],
                     axis=1).reshape(-1)

    h = x
    for lp in params["layers"]:
        wcat = jnp.concatenate(
            [lp["Wq"], lp["Wk"], lp["Wv"], lp["Ws"]], axis=1)
        bcat = jnp.concatenate(
            [lp["bq"], lp["bk"], lp["bv"], lp["bs"]]).reshape(1, 4 * D)
        wb = lp["Wb"][:, 0]
        wba = (wb[0:D] + wb[2 * D:3 * D]).reshape(1, D)
        wbb = (wb[D:2 * D] - wb[2 * D:3 * D]).reshape(1, D)

        qkvx = _proj(h, wcat, bcat)
        e = _e_matmul(edge_attr, lp["We"])
        parts = _sc_edge(qkvx, e, comb, n)
        den3 = parts[:, NPAD:NPAD + DR].reshape(NC, DR * 32, H)
        h = _gate(parts, den3, qkvx, wba, wbb, n)

    bn = 1000
    bidx3 = batch_index.astype(jnp.float32).reshape(n // bn, 1, bn)
    return _pool(h, bidx3, params["ro_W1"], params["ro_b1"].reshape(1, 32),
                 params["ro_W2"], params["ro_b2"].reshape(1, 10))
